# Initial kernel scaffold; baseline (speedup 1.0000x reference)
import jax
import jax.numpy as jnp
from jax.experimental import pallas as pl


def kernel(scores):
    raise NotImplementedError("write your pallas kernel here")


# speedup vs baseline: 4.8330x; 4.8330x over previous
import functools
import jax
import jax.numpy as jnp
from jax import lax
from jax.experimental import pallas as pl
from jax.experimental.pallas import tpu as pltpu
from jax.experimental.pallas import tpu_sc as plsc

_K = 1024
_N = 32768
_ROWS = 128
_L = 16
_NB = 256
_NW = 32
_RPW = _ROWS // _NW
_NV = _N // _L

_MININT = -2147483648
_NEG_INF = float('-inf')


def _keys_of(x):
    xz = jnp.where(x == 0.0, jnp.float32(0.0), x)
    b = lax.bitcast_convert_type(xz, jnp.int32)
    key = jnp.where(b < 0, b ^ 0x7FFFFFFF, b)
    k2 = key ^ _MININT
    return key, k2


def _iota():
    return lax.broadcasted_iota(jnp.int32, (_L,), 0)


def _process_hist(hist_ref, kp):
    iota = _iota()
    acc = jnp.zeros((_L,), jnp.int32)
    found = jnp.zeros((_L,), jnp.bool_)
    B = jnp.zeros((_L,), jnp.int32)
    mB = jnp.zeros((_L,), jnp.int32)
    kp_new = kp

    def j_body(jj, carry):
        acc, found, B, mB, kp_new = carry
        j = 15 - jj
        t = hist_ref[pl.ds(16 * j, _L)]
        for l in range(1, _L):
            t = t + hist_ref[pl.ds(l * _NB + 16 * j, _L)]
        rv = lax.rev(t, (0,))
        cs = plsc.cumsum(rv)
        full = cs + acc
        hitm = full >= kp
        anyhit = jnp.any(hitm)
        p = plsc.all_reduce_ffs(hitm)
        onehot = iota == p
        cab = jnp.sum(jnp.where(onehot, full - rv, 0))
        cb = jnp.sum(jnp.where(onehot, rv, 0))
        do = anyhit & (~found)
        B = jnp.where(do, 16 * j + 15 - p, B)
        kp_new = jnp.where(do, kp - cab, kp_new)
        mB = jnp.where(do, cb, mB)
        found = found | anyhit
        acc = acc + jnp.sum(t)
        return acc, found, B, mB, kp_new

    acc, found, B, mB, kp_new = lax.fori_loop(
        0, 16, j_body, (acc, found, B, mB, kp_new))
    return B, kp_new, mB


def _zero_hist(hist_ref):
    def z_body(i, _):
        hist_ref[pl.ds(i * _L, _L)] = jnp.zeros((_L,), jnp.int32)
        return 0

    lax.fori_loop(0, (_NB * _L) // _L, z_body, 0)


def _sc_body(scores_hbm, out_hbm, row_v, cand_v, hist_v):
    wid = lax.axis_index("s") * 2 + lax.axis_index("c")
    iota = _iota()
    lane_base = iota * _NB
    ones = jnp.ones((_L,), jnp.int32)

    def do_row(rr, _):
        row = wid * _RPW + rr
        pltpu.sync_copy(scores_hbm.at[row], row_v)

        _zero_hist(hist_v)

        def l1_body(i, _):
            x = row_v[pl.ds(i * _L, _L)]
            _, k2 = _keys_of(x)
            bucket = lax.shift_right_logical(k2, 24)
            plsc.addupdate_scatter(hist_v, [lane_base + bucket], ones)
            return 0

        lax.fori_loop(0, _NV, l1_body, 0, unroll=8)

        kp0 = jnp.full((_L,), jnp.int32(_K))
        B1, kp, m1 = _process_hist(hist_v, kp0)

        def p2_body(i, w):
            x = row_v[pl.ds(i * _L, _L)]
            _, k2 = _keys_of(x)
            bt = lax.shift_right_logical(k2, 24)
            row_v[pl.ds(i * _L, _L)] = jnp.where(bt > B1, _NEG_INF, x)
            match = bt == B1
            cs = plsc.cumsum(match.astype(jnp.int32))
            tgt = w + cs - 1
            plsc.store_scatter(cand_v, [tgt], i * _L + iota, mask=match)
            return w + plsc.all_reduce_population_count(match)

        lax.fori_loop(0, _NV, p2_body, jnp.zeros((_L,), jnp.int32),
                      unroll=8)

        nv_cand = (m1 + (_L - 1)) // _L
        nvc = jnp.max(nv_cand)

        def do_level(lvl, carry):
            prefix, kp = carry
            shift = 24 - 8 * lvl
            shift_hi = 32 - 8 * lvl
            _zero_hist(hist_v)

            def acc_body(i, _):
                base = i * _L
                tail = (base + iota) < m1
                ci = cand_v[pl.ds(base, _L)]
                x = plsc.load_gather(row_v, [ci], mask=tail)
                _, k2 = _keys_of(x)
                match = (lax.shift_right_logical(k2, shift_hi)
                         == prefix) & tail
                bucket = jnp.bitwise_and(
                    lax.shift_right_logical(k2, shift), _NB - 1)
                plsc.addupdate_scatter(hist_v, [lane_base + bucket],
                                       ones, mask=match)
                return 0

            lax.fori_loop(0, nvc, acc_body, 0)
            Bl, kp, _ = _process_hist(hist_v, kp)
            return (prefix * _NB + Bl), kp

        prefix, kp = lax.fori_loop(1, 4, do_level, (B1, kp))
        t_k2 = prefix
        T = t_k2 ^ _MININT
        need = kp

        def fix_body(i, eqseen):
            base = i * _L
            tail = (base + iota) < m1
            ci = cand_v[pl.ds(base, _L)]
            x = plsc.load_gather(row_v, [ci], mask=tail)
            key, _ = _keys_of(x)
            gtm = (key > T) & tail
            eqm = (key == T) & tail
            cs = plsc.cumsum(eqm.astype(jnp.int32))
            sel = eqm & ((eqseen + cs) <= need)
            plsc.store_scatter(row_v, [ci],
                               jnp.full((_L,), _NEG_INF, jnp.float32),
                               mask=gtm | sel)
            return eqseen + plsc.all_reduce_population_count(eqm)

        lax.fori_loop(0, nvc, fix_body, jnp.zeros((_L,), jnp.int32))

        pltpu.sync_copy(row_v, out_hbm.at[row])
        return 0

    lax.fori_loop(0, _RPW, do_row, 0)


@jax.jit
def kernel(scores):
    mesh = plsc.VectorSubcoreMesh(core_axis_name="c", subcore_axis_name="s", num_cores=2, num_subcores=16)
    f = pl.kernel(
        _sc_body,
        out_type=jax.ShapeDtypeStruct((_ROWS, _N), jnp.float32),
        mesh=mesh,
        compiler_params=pltpu.CompilerParams(needs_layout_passes=False),
        scratch_types=[
            pltpu.VMEM((_N,), jnp.float32),
            pltpu.VMEM((_N,), jnp.int32),
            pltpu.VMEM((_NB * _L,), jnp.int32),
        ],
    )
    return f(scores)
